# Initial kernel scaffold; baseline (speedup 1.0000x reference)
import jax
import jax.numpy as jnp
from jax.experimental import pallas as pl


def kernel(x, W_base, W_router, A, B):
    raise NotImplementedError("write your pallas kernel here")


# speedup vs baseline: 3.3416x; 3.3416x over previous
import functools

import jax
import jax.numpy as jnp
from jax.experimental import pallas as pl

N_TOKENS = 4096
D_IN = 1024
D_OUT = 1024
RANK = 8
NUM_EXPERTS = 64
TOP_K = 2
LORA_ALPHA = 32.0
_SCALING = LORA_ALPHA / RANK

_BN = 512


def _moe_lora_kernel(x_ref, wbt_ref, wrt_ref, at_ref, bf_ref, o_ref):
    xb = x_ref[:]
    logits = jnp.dot(xb, wrt_ref[:], preferred_element_type=jnp.float32)
    idx1 = jnp.argmax(logits, axis=-1)
    m1 = jnp.max(logits, axis=-1)
    eiota = jax.lax.broadcasted_iota(jnp.int32, logits.shape, 1)
    masked = jnp.where(eiota == idx1[:, None], -jnp.inf, logits)
    idx2 = jnp.argmax(masked, axis=-1)
    m2 = jnp.max(masked, axis=-1)
    g2 = 1.0 / (1.0 + jnp.exp(m1 - m2))
    g1 = 1.0 - g2

    h = jnp.dot(xb, at_ref[:], preferred_element_type=jnp.float32)
    col_expert = jax.lax.broadcasted_iota(jnp.int32, h.shape, 1) // RANK
    ge = jnp.where(col_expert == idx1[:, None], g1[:, None], 0.0) + jnp.where(
        col_expert == idx2[:, None], g2[:, None], 0.0
    )
    hw = h * ge
    lora = jnp.dot(hw, bf_ref[:], preferred_element_type=jnp.float32)
    base = jnp.dot(xb, wbt_ref[:], preferred_element_type=jnp.float32)
    o_ref[:] = base + _SCALING * lora


@functools.partial(jax.jit, static_argnames=())
def kernel(x, W_base, W_router, A, B):
    wbt = W_base.T
    wrt = W_router.T
    at = A.reshape(NUM_EXPERTS * RANK, D_IN).T
    bf = B.transpose(0, 2, 1).reshape(NUM_EXPERTS * RANK, D_OUT)

    grid = (N_TOKENS // _BN,)
    return pl.pallas_call(
        _moe_lora_kernel,
        grid=grid,
        in_specs=[
            pl.BlockSpec((_BN, D_IN), lambda i: (i, 0)),
            pl.BlockSpec((D_IN, D_OUT), lambda i: (0, 0)),
            pl.BlockSpec((D_IN, NUM_EXPERTS), lambda i: (0, 0)),
            pl.BlockSpec((D_IN, NUM_EXPERTS * RANK), lambda i: (0, 0)),
            pl.BlockSpec((NUM_EXPERTS * RANK, D_OUT), lambda i: (0, 0)),
        ],
        out_specs=pl.BlockSpec((_BN, D_OUT), lambda i: (i, 0)),
        out_shape=jax.ShapeDtypeStruct((N_TOKENS, D_OUT), jnp.float32),
    )(x, wbt, wrt, at, bf)
